# baseline (speedup 1.0000x reference)
import dataclasses
import functools

import numpy as np
import jax
import jax.numpy as jnp
from jax import lax
from jax.experimental import pallas as pl
from jax.experimental.pallas import tpu as pltpu
from jax.experimental.pallas import tpu_sc as plsc

B = 1024
D = 2048
OUT = 16000
N1 = 128
N2 = 125
L = 128
G = B // L
EPS = 1e-9

_k1 = np.arange(N1)
_a1 = 2.0 * np.pi * np.outer(_k1, _k1) / N1
_W1R = np.cos(_a1).astype(np.float32)
_W1I = (-np.sin(_a1)).astype(np.float32)
_k2 = np.arange(N2)
_a2 = 2.0 * np.pi * np.outer(_k2, _k2) / N2
_W2R = np.cos(_a2).astype(np.float32)
_W2I = (-np.sin(_a2)).astype(np.float32)
_IW2R = (np.cos(_a2) / N2).astype(np.float32)
_IW2I = (np.sin(_a2) / N2).astype(np.float32)
_ICOS = (np.cos(_a1) / N1).astype(np.float32)
_ISIN = (np.sin(_a1) / N1).astype(np.float32)

_PREC = lax.Precision.HIGHEST


def _sc_params():
    cp = pltpu.CompilerParams()
    if "needs_layout_passes" in pltpu.CompilerParams.__dataclass_fields__:
        cp = dataclasses.replace(cp, needs_layout_passes=False)
    return cp


def _dot(a, b):
    return lax.dot_general(a, b, (((1,), (0,)), ((), ())),
                           precision=_PREC,
                           preferred_element_type=jnp.float32)


HALF = 8000
SROWS = 8192


def _sketch_sc(x1t, s1, h1, x2t, s2, h2, interpret=False):
    del interpret
    mesh = plsc.VectorSubcoreMesh(core_axis_name="c", subcore_axis_name="s")
    out_sds = jax.ShapeDtypeStruct((G, OUT, L), jnp.float32)
    groups_per_core = G // 2
    feats_per_tile = D // 16
    zrows_per_tile = SROWS // 16

    @functools.partial(
        pl.kernel,
        out_type=[out_sds, out_sds],
        mesh=mesh,
        scratch_types=[
            pltpu.VMEM_SHARED((SROWS, L), jnp.float32),
            pltpu.VMEM((feats_per_tile, L), jnp.float32),
            pltpu.VMEM((feats_per_tile,), jnp.float32),
            pltpu.VMEM((feats_per_tile,), jnp.int32),
            pltpu.VMEM((feats_per_tile,), jnp.int32),
            pltpu.VMEM((feats_per_tile,), jnp.int32),
            pltpu.VMEM((256, L), jnp.float32),
        ],
        compiler_params=_sc_params(),
    )
    def k(x1_hbm, s1_hbm, h1_hbm, x2_hbm, s2_hbm, h2_hbm, o1_hbm, o2_hbm,
          spmem, xchunk, svec, hvec, idxraw, idxv, zbuf):
        c = lax.axis_index("c")
        t = lax.axis_index("s")
        zero16 = jnp.zeros((16,), jnp.float32)
        zeros16 = jnp.zeros((16,), jnp.int32)
        out_lo = t * 512

        @pl.loop(0, 256)
        def _(r):
            for j in range(L // 16):
                zbuf[r, pl.ds(j * 16, 16)] = zero16

        for (x_hbm, s_hbm, h_hbm, o_hbm) in (
                (x1_hbm, s1_hbm, h1_hbm, o1_hbm),
                (x2_hbm, s2_hbm, h2_hbm, o2_hbm)):
            pltpu.sync_copy(s_hbm.at[pl.ds(t * feats_per_tile,
                                           feats_per_tile)], svec)
            pltpu.sync_copy(h_hbm.at[pl.ds(t * feats_per_tile,
                                           feats_per_tile)], hvec)
            for j in range(feats_per_tile // 16):
                hh = hvec[pl.ds(j * 16, 16)]
                idxraw[pl.ds(j * 16, 16)] = (N2 * (hh % N1)) + (hh % N2)

            @pl.loop(0, groups_per_core)
            def _(gi):
                g = c * groups_per_core + gi
                pltpu.sync_copy(
                    x_hbm.at[pl.ds(t * feats_per_tile, feats_per_tile),
                             pl.ds(g * L, L)], xchunk)

                @pl.loop(0, feats_per_tile)
                def _(i):
                    sc = plsc.load_gather(svec, [zeros16 + i])
                    for j in range(L // 16):
                        sl = pl.ds(j * 16, 16)
                        xchunk[i, sl] = xchunk[i, sl] * sc

                @pl.loop(0, OUT // HALF)
                def _(half):
                    pltpu.sync_copy(zbuf, spmem.at[pl.ds(t * zrows_per_tile,
                                                         256)])
                    pltpu.sync_copy(zbuf,
                                    spmem.at[pl.ds(t * zrows_per_tile + 256,
                                                   256)])
                    lo = half * HALF
                    for j in range(feats_per_tile // 16):
                        sl = pl.ds(j * 16, 16)
                        rel = idxraw[sl] - lo
                        ok = (rel >= 0) & (rel < HALF)
                        idxv[sl] = jnp.where(ok, rel, HALF)
                    plsc.subcore_barrier()
                    pltpu.sync_copy(xchunk, spmem.at[idxv], add=True)
                    plsc.subcore_barrier()
                    @pl.when(t < 15)
                    def _():
                        pltpu.sync_copy(
                            spmem.at[pl.ds(out_lo, 512)],
                            o_hbm.at[g, pl.ds(lo + out_lo, 512)])

                    @pl.when(t == 15)
                    def _():
                        pltpu.sync_copy(
                            spmem.at[pl.ds(15 * 512, 320)],
                            o_hbm.at[g, pl.ds(lo + 15 * 512, 320)])

                    plsc.subcore_barrier()

    return k(x1t, s1, h1, x2t, s2, h2)


def _tc1(S1v, S2v, interpret=False):
    w1r = jnp.asarray(_W1R)
    w1i = jnp.asarray(_W1I)
    in_spec = pl.BlockSpec((1, N1, 1, 1, L), lambda g, m: (g, 0, m, 0, 0))
    w_spec = pl.BlockSpec((N1, N1), lambda g, m: (0, 0))
    out_spec = pl.BlockSpec((1, 1, N1, 1, L), lambda g, m: (g, m, 0, 0, 0))
    out_sds = jax.ShapeDtypeStruct((G, N2, N1, 1, L), jnp.float32)

    def body(s1_ref, s2_ref, wr_ref, wi_ref, f1r, f1i, f2r, f2i):
        wr = wr_ref[...]
        wi = wi_ref[...]
        for s_ref, fr, fi in ((s1_ref, f1r, f1i), (s2_ref, f2r, f2i)):
            s = s_ref[0, :, 0, 0, :]
            fr[0, 0, :, 0, :] = _dot(wr, s)
            fi[0, 0, :, 0, :] = _dot(wi, s)

    return pl.pallas_call(
        body,
        grid=(G, N2),
        in_specs=[in_spec, in_spec, w_spec, w_spec],
        out_specs=[out_spec] * 4,
        out_shape=[out_sds] * 4,
        interpret=interpret,
    )(S1v, S2v, w1r, w1i)


def _tc2(f1r, f1i, f2r, f2i, interpret=False):
    w2r = jnp.asarray(_W2R)
    w2i = jnp.asarray(_W2I)
    iw2r = jnp.asarray(_IW2R)
    iw2i = jnp.asarray(_IW2I)
    in_spec = pl.BlockSpec((1, N2, 1, 1, L), lambda g, k: (g, 0, k, 0, 0))
    w_spec = pl.BlockSpec((N2, N2), lambda g, k: (0, 0))
    out_spec = pl.BlockSpec((1, 1, N2, 1, L), lambda g, k: (g, k, 0, 0, 0))
    out_sds = jax.ShapeDtypeStruct((G, N1, N2, 1, L), jnp.float32)

    def body(a_r_ref, a_i_ref, b_r_ref, b_i_ref,
             w2r_ref, w2i_ref, iw2r_ref, iw2i_ref, rr, ri):
        w_r = w2r_ref[...]
        w_i = w2i_ref[...]
        ar = a_r_ref[0, :, 0, 0, :]
        ai = a_i_ref[0, :, 0, 0, :]
        br = b_r_ref[0, :, 0, 0, :]
        bi = b_i_ref[0, :, 0, 0, :]
        Ar = _dot(w_r, ar) - _dot(w_i, ai)
        Ai = _dot(w_r, ai) + _dot(w_i, ar)
        Br = _dot(w_r, br) - _dot(w_i, bi)
        Bi = _dot(w_r, bi) + _dot(w_i, br)
        Pr = Ar * Br - Ai * Bi
        Pi = Ar * Bi + Ai * Br
        v_r = iw2r_ref[...]
        v_i = iw2i_ref[...]
        rr[0, 0, :, 0, :] = _dot(v_r, Pr) - _dot(v_i, Pi)
        ri[0, 0, :, 0, :] = _dot(v_r, Pi) + _dot(v_i, Pr)

    return pl.pallas_call(
        body,
        grid=(G, N1),
        in_specs=[in_spec] * 4 + [w_spec] * 4,
        out_specs=[out_spec] * 2,
        out_shape=[out_sds] * 2,
        interpret=interpret,
    )(f1r, f1i, f2r, f2i, w2r, w2i, iw2r, iw2i)


def _tc3(rr, ri, interpret=False):
    icos = jnp.asarray(_ICOS)
    isin = jnp.asarray(_ISIN)
    in_spec = pl.BlockSpec((1, N1, 1, 1, L), lambda g, m: (g, 0, m, 0, 0))
    w_spec = pl.BlockSpec((N1, N1), lambda g, m: (0, 0))
    y_spec = pl.BlockSpec((1, N1, 1, 1, L), lambda g, m: (g, 0, m, 0, 0))
    p_spec = pl.BlockSpec((1, 1, 8, L), lambda g, m: (g, m, 0, 0))
    y_sds = jax.ShapeDtypeStruct((G, N1, N2, 1, L), jnp.float32)
    p_sds = jax.ShapeDtypeStruct((G, N2, 8, L), jnp.float32)

    def body(rr_ref, ri_ref, c_ref, s_ref, y_ref, p_ref):
        cm = c_ref[...]
        sm = s_ref[...]
        zr = rr_ref[0, :, 0, 0, :]
        zi = ri_ref[0, :, 0, 0, :]
        c2 = _dot(cm, zr) - _dot(sm, zi)
        y = jnp.sign(c2) * jnp.sqrt(jnp.abs(c2) + EPS)
        y_ref[0, :, 0, 0, :] = y
        psum = jnp.sum(y * y, axis=0, keepdims=True)
        p_ref[0, 0, :, :] = jnp.broadcast_to(psum, (8, L))

    return pl.pallas_call(
        body,
        grid=(G, N2),
        in_specs=[in_spec, in_spec, w_spec, w_spec],
        out_specs=[y_spec, p_spec],
        out_shape=[y_sds, p_sds],
        interpret=interpret,
    )(rr, ri, icos, isin)


def _tc4(p, interpret=False):
    in_spec = pl.BlockSpec((G, N2, 8, L), lambda: (0, 0, 0, 0))
    out_spec = pl.BlockSpec((G, 8, L), lambda: (0, 0, 0))
    out_sds = jax.ShapeDtypeStruct((G, 8, L), jnp.float32)

    def body(p_ref, inv_ref):
        for g in range(G):
            pv = p_ref[g, :, 0, :]
            sg = jnp.sum(pv, axis=0, keepdims=True)
            inv = 1.0 / jnp.maximum(jnp.sqrt(sg), EPS)
            inv_ref[g, :, :] = jnp.broadcast_to(inv, (8, L))

    return pl.pallas_call(
        body,
        grid=(),
        in_specs=[in_spec],
        out_specs=out_spec,
        out_shape=out_sds,
        interpret=interpret,
    )(p)


def _final_sc(yf, inv, interpret=False):
    del interpret
    mesh = plsc.VectorSubcoreMesh(core_axis_name="c", subcore_axis_name="s")
    out_sds = jax.ShapeDtypeStruct((B, OUT), jnp.float32)
    nblocks = G * N2

    @functools.partial(
        pl.kernel,
        out_type=out_sds,
        mesh=mesh,
        scratch_types=[
            pltpu.VMEM((L, L), jnp.float32),
            pltpu.VMEM((L, L), jnp.float32),
            pltpu.VMEM((L,), jnp.int32),
            pltpu.VMEM((L,), jnp.float32),
        ],
        compiler_params=_sc_params(),
    )
    def k(y_hbm, inv_hbm, o_hbm, rowsbuf, outbuf, idxv, invvec):
        c = lax.axis_index("c")
        t = lax.axis_index("s")
        wid = c * 16 + t
        iota = lax.iota(jnp.int32, 16)

        @pl.loop(wid, nblocks, step=32)
        def _(bb):
            g = bb // N2
            kb = bb - g * N2
            for jg in range(L // 16):
                jj = jg * 16 + iota
                idxv[pl.ds(jg * 16, 16)] = (
                    g * OUT + N2 * jj + (3 * kb + jj) % N2)
            pltpu.sync_copy(y_hbm.at[idxv], rowsbuf)
            pltpu.sync_copy(inv_hbm.at[g, 0], invvec)

            zeros16 = jnp.zeros((16,), jnp.int32)

            @pl.loop(0, L)
            def _(l):
                lvec = zeros16 + l
                sc = plsc.load_gather(invvec, [lvec])
                for jg in range(L // 16):
                    ridx = jg * 16 + iota
                    v = plsc.load_gather(rowsbuf, [ridx, lvec])
                    outbuf[l, pl.ds(jg * 16, 16)] = v * sc

            pltpu.sync_copy(outbuf,
                            o_hbm.at[pl.ds(g * L, L), pl.ds(kb * L, L)])

    return k(yf, inv)


def kernel(x1, x2, s1, s2, h1, h2):
    x1t = jnp.transpose(x1)
    x2t = jnp.transpose(x2)
    h1 = h1.astype(jnp.int32)
    h2 = h2.astype(jnp.int32)

    S1, S2 = _sketch_sc(x1t, s1, h1, x2t, s2, h2)
    S1v = S1.reshape(G, N1, N2, 1, L)
    S2v = S2.reshape(G, N1, N2, 1, L)
    f1r, f1i, f2r, f2i = _tc1(S1v, S2v)
    rr, ri = _tc2(f1r, f1i, f2r, f2i)
    y, p = _tc3(rr, ri)
    inv = _tc4(p)
    return _final_sc(y.reshape(G * OUT, L), inv)


# speedup vs baseline: 1.9216x; 1.9216x over previous
import dataclasses
import functools

import numpy as np
import jax
import jax.numpy as jnp
from jax import lax
from jax.experimental import pallas as pl
from jax.experimental.pallas import tpu as pltpu
from jax.experimental.pallas import tpu_sc as plsc

B = 1024
D = 2048
OUT = 16000
N1 = 128
N2 = 125
N2P = 128
K1 = 65
K1P = 72
OUTP = N1 * N2P
L = 128
G = B // L
EPS = 1e-9

_k1 = np.arange(N1)
_a1 = 2.0 * np.pi * np.outer(_k1, _k1) / N1
_W1R = np.zeros((K1P, N1), np.float32)
_W1I = np.zeros((K1P, N1), np.float32)
_W1R[:K1] = np.cos(_a1)[:K1]
_W1I[:K1] = -np.sin(_a1)[:K1]
_k2 = np.arange(N2)
_a2 = 2.0 * np.pi * np.outer(_k2, _k2) / N2
_W2R = np.zeros((N2P, N2P), np.float32)
_W2I = np.zeros((N2P, N2P), np.float32)
_W2R[:N2, :N2] = np.cos(_a2)
_W2I[:N2, :N2] = -np.sin(_a2)
_IW2R = np.zeros((N2P, N2P), np.float32)
_IW2I = np.zeros((N2P, N2P), np.float32)
_IW2R[:N2, :N2] = np.cos(_a2) / N2
_IW2I[:N2, :N2] = np.sin(_a2) / N2
_c = np.where((_k1 == 0) | (_k1 == 64), 1.0, 2.0)[:K1]
_REC_A = np.zeros((N1, K1P), np.float32)
_REC_B = np.zeros((N1, K1P), np.float32)
_REC_A[:, :K1] = _c * np.cos(_a1[:, :K1]) / N1
_REC_B[:, :K1] = -_c * np.sin(_a1[:, :K1]) / N1

_PREC = lax.Precision.HIGHEST


def _dot(a, b):
    return lax.dot_general(a, b, (((1,), (0,)), ((), ())),
                           precision=_PREC,
                           preferred_element_type=jnp.float32)


def _sc_params():
    cp = pltpu.CompilerParams()
    if "needs_layout_passes" in pltpu.CompilerParams.__dataclass_fields__:
        cp = dataclasses.replace(cp, needs_layout_passes=False)
    return cp


HALF = OUTP // 2
SROWS = 8448


def _sketch_sc(x1t, s1, h1, x2t, s2, h2):
    mesh = plsc.VectorSubcoreMesh(core_axis_name="c", subcore_axis_name="s")
    out_sds = jax.ShapeDtypeStruct((G, OUTP, L), jnp.float32)
    groups_per_core = G // 2
    feats_per_tile = D // 16
    zrows_per_tile = SROWS // 16
    orows_per_tile = HALF // 16

    @functools.partial(
        pl.kernel,
        out_type=[out_sds, out_sds],
        mesh=mesh,
        scratch_types=[
            pltpu.VMEM_SHARED((SROWS, L), jnp.float32),
            pltpu.VMEM((feats_per_tile, L), jnp.float32),
            pltpu.VMEM((feats_per_tile,), jnp.float32),
            pltpu.VMEM((feats_per_tile,), jnp.int32),
            pltpu.VMEM((feats_per_tile,), jnp.int32),
            pltpu.VMEM((feats_per_tile,), jnp.int32),
            pltpu.VMEM((264, L), jnp.float32),
        ],
        compiler_params=_sc_params(),
    )
    def k(x1_hbm, s1_hbm, h1_hbm, x2_hbm, s2_hbm, h2_hbm, o1_hbm, o2_hbm,
          spmem, xchunk, svec, hvec, idxraw, idxv, zbuf):
        c = lax.axis_index("c")
        t = lax.axis_index("s")
        zero16 = jnp.zeros((16,), jnp.float32)
        zeros16 = jnp.zeros((16,), jnp.int32)

        @pl.loop(0, 264)
        def _(r):
            for j in range(L // 16):
                zbuf[r, pl.ds(j * 16, 16)] = zero16

        for (x_hbm, s_hbm, h_hbm, o_hbm) in (
                (x1_hbm, s1_hbm, h1_hbm, o1_hbm),
                (x2_hbm, s2_hbm, h2_hbm, o2_hbm)):
            pltpu.sync_copy(s_hbm.at[pl.ds(t * feats_per_tile,
                                           feats_per_tile)], svec)
            pltpu.sync_copy(h_hbm.at[pl.ds(t * feats_per_tile,
                                           feats_per_tile)], hvec)
            for j in range(feats_per_tile // 16):
                hh = hvec[pl.ds(j * 16, 16)]
                idxraw[pl.ds(j * 16, 16)] = (N2P * (hh % N1)) + (hh % N2)

            @pl.loop(0, groups_per_core)
            def _(gi):
                g = c * groups_per_core + gi
                pltpu.sync_copy(
                    x_hbm.at[pl.ds(t * feats_per_tile, feats_per_tile),
                             pl.ds(g * L, L)], xchunk)

                @pl.loop(0, feats_per_tile)
                def _(i):
                    sc = plsc.load_gather(svec, [zeros16 + i])
                    for j in range(L // 16):
                        sl = pl.ds(j * 16, 16)
                        xchunk[i, sl] = xchunk[i, sl] * sc

                @pl.loop(0, OUTP // HALF)
                def _(half):
                    pltpu.sync_copy(zbuf, spmem.at[pl.ds(t * zrows_per_tile,
                                                         264)])
                    pltpu.sync_copy(zbuf,
                                    spmem.at[pl.ds(t * zrows_per_tile + 264,
                                                   264)])
                    lo = half * HALF
                    for j in range(feats_per_tile // 16):
                        sl = pl.ds(j * 16, 16)
                        rel = idxraw[sl] - lo
                        ok = (rel >= 0) & (rel < HALF)
                        idxv[sl] = jnp.where(ok, rel, HALF)
                    plsc.subcore_barrier()
                    pltpu.sync_copy(xchunk, spmem.at[idxv], add=True)
                    plsc.subcore_barrier()
                    pltpu.sync_copy(
                        spmem.at[pl.ds(t * orows_per_tile, orows_per_tile)],
                        o_hbm.at[g, pl.ds(lo + t * orows_per_tile,
                                          orows_per_tile)])
                    plsc.subcore_barrier()

    return k(x1t, s1, h1, x2t, s2, h2)


_T1 = 16


def _tc1(S1v, S2v, interpret=False):
    w1r = jnp.asarray(_W1R)
    w1i = jnp.asarray(_W1I)
    in_spec = pl.BlockSpec((1, N1, _T1, L), lambda g, m: (g, 0, m, 0))
    w_spec = pl.BlockSpec((K1P, N1), lambda g, m: (0, 0))
    out_spec = pl.BlockSpec((1, _T1, K1P, L), lambda g, m: (g, m, 0, 0))
    out_sds = jax.ShapeDtypeStruct((G, N2P, K1P, L), jnp.float32)

    def body(s1_ref, s2_ref, wr_ref, wi_ref, f1r, f1i, f2r, f2i):
        wr = wr_ref[...]
        wi = wi_ref[...]
        for s_ref, fr, fi in ((s1_ref, f1r, f1i), (s2_ref, f2r, f2i)):
            for t in range(_T1):
                s = s_ref[0, :, t, :]
                fr[0, t, :, :] = _dot(wr, s)
                fi[0, t, :, :] = _dot(wi, s)

    return pl.pallas_call(
        body,
        grid=(G, N2P // _T1),
        in_specs=[in_spec, in_spec, w_spec, w_spec],
        out_specs=[out_spec] * 4,
        out_shape=[out_sds] * 4,
        interpret=interpret,
    )(S1v, S2v, w1r, w1i)


_T2 = 8


def _tc2(f1r, f1i, f2r, f2i, interpret=False):
    w2r = jnp.asarray(_W2R)
    w2i = jnp.asarray(_W2I)
    iw2r = jnp.asarray(_IW2R)
    iw2i = jnp.asarray(_IW2I)
    in_spec = pl.BlockSpec((1, N2P, _T2, L), lambda g, k: (g, 0, k, 0))
    w_spec = pl.BlockSpec((N2P, N2P), lambda g, k: (0, 0))
    out_spec = pl.BlockSpec((1, _T2, N2P, L), lambda g, k: (g, k, 0, 0))
    out_sds = jax.ShapeDtypeStruct((G, K1P, N2P, L), jnp.float32)

    def body(a_r_ref, a_i_ref, b_r_ref, b_i_ref,
             w2r_ref, w2i_ref, iw2r_ref, iw2i_ref, rr, ri):
        w_r = w2r_ref[...]
        w_i = w2i_ref[...]
        v_r = iw2r_ref[...]
        v_i = iw2i_ref[...]
        for t in range(_T2):
            ar = a_r_ref[0, :, t, :]
            ai = a_i_ref[0, :, t, :]
            br = b_r_ref[0, :, t, :]
            bi = b_i_ref[0, :, t, :]
            Ar = _dot(w_r, ar) - _dot(w_i, ai)
            Ai = _dot(w_r, ai) + _dot(w_i, ar)
            Br = _dot(w_r, br) - _dot(w_i, bi)
            Bi = _dot(w_r, bi) + _dot(w_i, br)
            Pr = Ar * Br - Ai * Bi
            Pi = Ar * Bi + Ai * Br
            rr[0, t, :, :] = _dot(v_r, Pr) - _dot(v_i, Pi)
            ri[0, t, :, :] = _dot(v_r, Pi) + _dot(v_i, Pr)

    return pl.pallas_call(
        body,
        grid=(G, K1P // _T2),
        in_specs=[in_spec] * 4 + [w_spec] * 4,
        out_specs=[out_spec] * 2,
        out_shape=[out_sds] * 2,
        interpret=interpret,
    )(f1r, f1i, f2r, f2i, w2r, w2i, iw2r, iw2i)


_T3 = 16


def _tc3(rr, ri, interpret=False):
    reca = jnp.asarray(_REC_A)
    recb = jnp.asarray(_REC_B)
    in_spec = pl.BlockSpec((1, K1P, _T3, L), lambda g, m: (g, 0, m, 0))
    w_spec = pl.BlockSpec((N1, K1P), lambda g, m: (0, 0))
    y_spec = pl.BlockSpec((1, N1, _T3, L), lambda g, m: (g, 0, m, 0))
    p_spec = pl.BlockSpec((1, _T3, 8, L), lambda g, m: (g, m, 0, 0))
    y_sds = jax.ShapeDtypeStruct((G, N1, N2P, L), jnp.float32)
    p_sds = jax.ShapeDtypeStruct((G, N2P, 8, L), jnp.float32)

    def body(rr_ref, ri_ref, a_ref, b_ref, y_ref, p_ref):
        am = a_ref[...]
        bm = b_ref[...]
        for t in range(_T3):
            zr = rr_ref[0, :, t, :]
            zi = ri_ref[0, :, t, :]
            c2 = _dot(am, zr) + _dot(bm, zi)
            y = jnp.sign(c2) * jnp.sqrt(jnp.abs(c2) + EPS)
            y_ref[0, :, t, :] = y
            psum = jnp.sum(y * y, axis=0, keepdims=True)
            p_ref[0, t, :, :] = jnp.broadcast_to(psum, (8, L))

    return pl.pallas_call(
        body,
        grid=(G, N2P // _T3),
        in_specs=[in_spec, in_spec, w_spec, w_spec],
        out_specs=[y_spec, p_spec],
        out_shape=[y_sds, p_sds],
        interpret=interpret,
    )(rr, ri, reca, recb)


def _tc4(p, interpret=False):
    in_spec = pl.BlockSpec((G, N2P, 8, L), lambda: (0, 0, 0, 0))
    out_spec = pl.BlockSpec((G, 8, L), lambda: (0, 0, 0))
    out_sds = jax.ShapeDtypeStruct((G, 8, L), jnp.float32)

    def body(p_ref, inv_ref):
        for g in range(G):
            pv = p_ref[g, :, 0, :]
            sg = jnp.sum(pv, axis=0, keepdims=True)
            inv = 1.0 / jnp.maximum(jnp.sqrt(sg), EPS)
            inv_ref[g, :, :] = jnp.broadcast_to(inv, (8, L))

    return pl.pallas_call(
        body,
        grid=(),
        in_specs=[in_spec],
        out_specs=out_spec,
        out_shape=out_sds,
        interpret=interpret,
    )(p)


def _final_sc(yf):
    mesh = plsc.VectorSubcoreMesh(core_axis_name="c", subcore_axis_name="s")
    out_sds = jax.ShapeDtypeStruct((G, N2, L, L), jnp.float32)
    nblocks = G * N2

    @functools.partial(
        pl.kernel,
        out_type=out_sds,
        mesh=mesh,
        scratch_types=[
            pltpu.VMEM((L, L), jnp.float32),
            pltpu.VMEM((L,), jnp.int32),
        ],
        compiler_params=_sc_params(),
    )
    def k(y_hbm, o_hbm, rowsbuf, idxv):
        c = lax.axis_index("c")
        t = lax.axis_index("s")
        wid = c * 16 + t
        iota = lax.iota(jnp.int32, 16)

        @pl.loop(wid, nblocks, step=32)
        def _(bb):
            g = bb // N2
            kb = bb - g * N2
            for jg in range(L // 16):
                jj = jg * 16 + iota
                idxv[pl.ds(jg * 16, 16)] = (
                    g * OUTP + N2P * jj + (3 * kb + jj) % N2)
            pltpu.sync_copy(y_hbm.at[idxv], rowsbuf)
            pltpu.sync_copy(rowsbuf, o_hbm.at[g, kb])

    return k(yf)


def _tc5(p2, inv, interpret=False):
    in_spec = pl.BlockSpec((1, 1, L, L), lambda g, kb: (g, kb, 0, 0))
    inv_spec = pl.BlockSpec((1, 8, L), lambda g, kb: (g, 0, 0))
    out_spec = pl.BlockSpec((L, L), lambda g, kb: (g, kb))
    out_sds = jax.ShapeDtypeStruct((B, OUT), jnp.float32)

    def body(p_ref, inv_ref, o_ref):
        tile = p_ref[0, 0, :, :]
        invrow = inv_ref[0, 0:1, :]
        o_ref[...] = jnp.transpose(tile * invrow, (1, 0))

    return pl.pallas_call(
        body,
        grid=(G, N2),
        in_specs=[in_spec, inv_spec],
        out_specs=out_spec,
        out_shape=out_sds,
        interpret=interpret,
    )(p2, inv)


def kernel(x1, x2, s1, s2, h1, h2):
    x1t = jnp.transpose(x1)
    x2t = jnp.transpose(x2)
    h1 = h1.astype(jnp.int32)
    h2 = h2.astype(jnp.int32)

    S1, S2 = _sketch_sc(x1t, s1, h1, x2t, s2, h2)
    S1v = S1.reshape(G, N1, N2P, L)
    S2v = S2.reshape(G, N1, N2P, L)
    f1r, f1i, f2r, f2i = _tc1(S1v, S2v)
    rr, ri = _tc2(f1r, f1i, f2r, f2i)
    y, p = _tc3(rr, ri)
    inv = _tc4(p)
    p2 = _final_sc(y.reshape(G * OUTP, L))
    return _tc5(p2, inv)


# speedup vs baseline: 2.6735x; 1.3913x over previous
import dataclasses
import functools

import numpy as np
import jax
import jax.numpy as jnp
from jax import lax
from jax.experimental import pallas as pl
from jax.experimental.pallas import tpu as pltpu
from jax.experimental.pallas import tpu_sc as plsc

B = 1024
D = 2048
OUT = 16000
N1 = 128
N2 = 125
N2P = 128
K1 = 65
K1P = 72
OUTP = N1 * N2P
L = 128
G = B // L
EPS = 1e-9

_k1 = np.arange(N1)
_a1 = 2.0 * np.pi * np.outer(_k1, _k1) / N1
_W1R = np.zeros((K1P, N1), np.float32)
_W1I = np.zeros((K1P, N1), np.float32)
_W1R[:K1] = np.cos(_a1)[:K1]
_W1I[:K1] = -np.sin(_a1)[:K1]
_k2 = np.arange(N2)
_a2 = 2.0 * np.pi * np.outer(_k2, _k2) / N2
_W2R = np.zeros((N2P, N2P), np.float32)
_W2I = np.zeros((N2P, N2P), np.float32)
_W2R[:N2, :N2] = np.cos(_a2)
_W2I[:N2, :N2] = -np.sin(_a2)
_IW2R = np.zeros((N2P, N2P), np.float32)
_IW2I = np.zeros((N2P, N2P), np.float32)
_IW2R[:N2, :N2] = np.cos(_a2) / N2
_IW2I[:N2, :N2] = np.sin(_a2) / N2
_c = np.where((_k1 == 0) | (_k1 == 64), 1.0, 2.0)[:K1]
_REC_A = np.zeros((N1, K1P), np.float32)
_REC_B = np.zeros((N1, K1P), np.float32)
_REC_A[:, :K1] = _c * np.cos(_a1[:, :K1]) / N1
_REC_B[:, :K1] = -_c * np.sin(_a1[:, :K1]) / N1

_PREC = lax.Precision.DEFAULT


def _dot(a, b):
    return lax.dot_general(a, b, (((1,), (0,)), ((), ())),
                           precision=_PREC,
                           preferred_element_type=jnp.float32)


def _sc_params():
    cp = pltpu.CompilerParams()
    if "needs_layout_passes" in pltpu.CompilerParams.__dataclass_fields__:
        cp = dataclasses.replace(cp, needs_layout_passes=False)
    return cp


HALF = OUTP // 2
SROWS = 8448


def _sketch_sc(x1t, s1, h1, x2t, s2, h2):
    mesh = plsc.VectorSubcoreMesh(core_axis_name="c", subcore_axis_name="s")
    out_sds = jax.ShapeDtypeStruct((G, OUTP, L), jnp.float32)
    groups_per_core = G // 2
    feats_per_tile = D // 16
    zrows_per_tile = SROWS // 16
    orows_per_tile = HALF // 16

    @functools.partial(
        pl.kernel,
        out_type=[out_sds, out_sds],
        mesh=mesh,
        scratch_types=[
            pltpu.VMEM_SHARED((SROWS, L), jnp.float32),
            pltpu.VMEM((feats_per_tile, L), jnp.float32),
            pltpu.VMEM((feats_per_tile,), jnp.float32),
            pltpu.VMEM((feats_per_tile,), jnp.int32),
            pltpu.VMEM((feats_per_tile,), jnp.int32),
            pltpu.VMEM((feats_per_tile,), jnp.int32),
            pltpu.VMEM((264, L), jnp.float32),
        ],
        compiler_params=_sc_params(),
    )
    def k(x1_hbm, s1_hbm, h1_hbm, x2_hbm, s2_hbm, h2_hbm, o1_hbm, o2_hbm,
          spmem, xchunk, svec, hvec, idxraw, idxv, zbuf):
        c = lax.axis_index("c")
        t = lax.axis_index("s")
        zero16 = jnp.zeros((16,), jnp.float32)
        zeros16 = jnp.zeros((16,), jnp.int32)

        @pl.loop(0, 264)
        def _(r):
            for j in range(L // 16):
                zbuf[r, pl.ds(j * 16, 16)] = zero16

        for (x_hbm, s_hbm, h_hbm, o_hbm) in (
                (x1_hbm, s1_hbm, h1_hbm, o1_hbm),
                (x2_hbm, s2_hbm, h2_hbm, o2_hbm)):
            pltpu.sync_copy(s_hbm.at[pl.ds(t * feats_per_tile,
                                           feats_per_tile)], svec)
            pltpu.sync_copy(h_hbm.at[pl.ds(t * feats_per_tile,
                                           feats_per_tile)], hvec)
            for j in range(feats_per_tile // 16):
                hh = hvec[pl.ds(j * 16, 16)]
                idxraw[pl.ds(j * 16, 16)] = (N2P * (hh % N1)) + (hh % N2)

            @pl.loop(0, groups_per_core)
            def _(gi):
                g = c * groups_per_core + gi
                pltpu.sync_copy(
                    x_hbm.at[pl.ds(t * feats_per_tile, feats_per_tile),
                             pl.ds(g * L, L)], xchunk)

                @pl.loop(0, feats_per_tile)
                def _(i):
                    sc = plsc.load_gather(svec, [zeros16 + i])
                    for j in range(L // 16):
                        sl = pl.ds(j * 16, 16)
                        xchunk[i, sl] = xchunk[i, sl] * sc

                @pl.loop(0, OUTP // HALF)
                def _(half):
                    pltpu.sync_copy(zbuf, spmem.at[pl.ds(t * zrows_per_tile,
                                                         264)])
                    pltpu.sync_copy(zbuf,
                                    spmem.at[pl.ds(t * zrows_per_tile + 264,
                                                   264)])
                    lo = half * HALF
                    for j in range(feats_per_tile // 16):
                        sl = pl.ds(j * 16, 16)
                        rel = idxraw[sl] - lo
                        ok = (rel >= 0) & (rel < HALF)
                        idxv[sl] = jnp.where(ok, rel, HALF)
                    plsc.subcore_barrier()
                    pltpu.sync_copy(xchunk, spmem.at[idxv], add=True)
                    plsc.subcore_barrier()
                    pltpu.sync_copy(
                        spmem.at[pl.ds(t * orows_per_tile, orows_per_tile)],
                        o_hbm.at[g, pl.ds(lo + t * orows_per_tile,
                                          orows_per_tile)])
                    plsc.subcore_barrier()

    return k(x1t, s1, h1, x2t, s2, h2)


_T1 = 16


def _tc1(S1v, S2v, interpret=False):
    w1r = jnp.asarray(_W1R)
    w1i = jnp.asarray(_W1I)
    in_spec = pl.BlockSpec((1, N1, _T1, L), lambda g, m: (g, 0, m, 0))
    w_spec = pl.BlockSpec((K1P, N1), lambda g, m: (0, 0))
    out_spec = pl.BlockSpec((1, _T1, K1P, L), lambda g, m: (g, m, 0, 0))
    out_sds = jax.ShapeDtypeStruct((G, N2P, K1P, L), jnp.float32)

    def body(s1_ref, s2_ref, wr_ref, wi_ref, f1r, f1i, f2r, f2i):
        wr = wr_ref[...]
        wi = wi_ref[...]
        for s_ref, fr, fi in ((s1_ref, f1r, f1i), (s2_ref, f2r, f2i)):
            for t in range(_T1):
                s = s_ref[0, :, t, :]
                fr[0, t, :, :] = _dot(wr, s)
                fi[0, t, :, :] = _dot(wi, s)

    return pl.pallas_call(
        body,
        grid=(G, N2P // _T1),
        in_specs=[in_spec, in_spec, w_spec, w_spec],
        out_specs=[out_spec] * 4,
        out_shape=[out_sds] * 4,
        interpret=interpret,
    )(S1v, S2v, w1r, w1i)


_T2 = 8


def _tc2(f1r, f1i, f2r, f2i, interpret=False):
    w2r = jnp.asarray(_W2R)
    w2i = jnp.asarray(_W2I)
    iw2r = jnp.asarray(_IW2R)
    iw2i = jnp.asarray(_IW2I)
    in_spec = pl.BlockSpec((1, N2P, _T2, L), lambda g, k: (g, 0, k, 0))
    w_spec = pl.BlockSpec((N2P, N2P), lambda g, k: (0, 0))
    out_spec = pl.BlockSpec((1, _T2, N2P, L), lambda g, k: (g, k, 0, 0))
    out_sds = jax.ShapeDtypeStruct((G, K1P, N2P, L), jnp.float32)

    def body(a_r_ref, a_i_ref, b_r_ref, b_i_ref,
             w2r_ref, w2i_ref, iw2r_ref, iw2i_ref, rr, ri):
        w_r = w2r_ref[...]
        w_i = w2i_ref[...]
        v_r = iw2r_ref[...]
        v_i = iw2i_ref[...]
        for t in range(_T2):
            ar = a_r_ref[0, :, t, :]
            ai = a_i_ref[0, :, t, :]
            br = b_r_ref[0, :, t, :]
            bi = b_i_ref[0, :, t, :]
            Ar = _dot(w_r, ar) - _dot(w_i, ai)
            Ai = _dot(w_r, ai) + _dot(w_i, ar)
            Br = _dot(w_r, br) - _dot(w_i, bi)
            Bi = _dot(w_r, bi) + _dot(w_i, br)
            Pr = Ar * Br - Ai * Bi
            Pi = Ar * Bi + Ai * Br
            rr[0, t, :, :] = _dot(v_r, Pr) - _dot(v_i, Pi)
            ri[0, t, :, :] = _dot(v_r, Pi) + _dot(v_i, Pr)

    return pl.pallas_call(
        body,
        grid=(G, K1P // _T2),
        in_specs=[in_spec] * 4 + [w_spec] * 4,
        out_specs=[out_spec] * 2,
        out_shape=[out_sds] * 2,
        interpret=interpret,
    )(f1r, f1i, f2r, f2i, w2r, w2i, iw2r, iw2i)


_T3 = 16


def _tc3(rr, ri, interpret=False):
    reca = jnp.asarray(_REC_A)
    recb = jnp.asarray(_REC_B)
    in_spec = pl.BlockSpec((1, K1P, _T3, L), lambda g, m: (g, 0, m, 0))
    w_spec = pl.BlockSpec((N1, K1P), lambda g, m: (0, 0))
    y_spec = pl.BlockSpec((1, N1, _T3, L), lambda g, m: (g, 0, m, 0))
    p_spec = pl.BlockSpec((1, _T3, 8, L), lambda g, m: (g, m, 0, 0))
    y_sds = jax.ShapeDtypeStruct((G, N1, N2P, L), jnp.float32)
    p_sds = jax.ShapeDtypeStruct((G, N2P, 8, L), jnp.float32)

    def body(rr_ref, ri_ref, a_ref, b_ref, y_ref, p_ref):
        am = a_ref[...]
        bm = b_ref[...]
        for t in range(_T3):
            zr = rr_ref[0, :, t, :]
            zi = ri_ref[0, :, t, :]
            c2 = _dot(am, zr) + _dot(bm, zi)
            y = jnp.sign(c2) * jnp.sqrt(jnp.abs(c2) + EPS)
            y_ref[0, :, t, :] = y
            psum = jnp.sum(y * y, axis=0, keepdims=True)
            p_ref[0, t, :, :] = jnp.broadcast_to(psum, (8, L))

    return pl.pallas_call(
        body,
        grid=(G, N2P // _T3),
        in_specs=[in_spec, in_spec, w_spec, w_spec],
        out_specs=[y_spec, p_spec],
        out_shape=[y_sds, p_sds],
        interpret=interpret,
    )(rr, ri, reca, recb)


def _tc4(p, interpret=False):
    in_spec = pl.BlockSpec((G, N2P, 8, L), lambda: (0, 0, 0, 0))
    out_spec = pl.BlockSpec((G, 8, L), lambda: (0, 0, 0))
    out_sds = jax.ShapeDtypeStruct((G, 8, L), jnp.float32)

    def body(p_ref, inv_ref):
        for g in range(G):
            pv = p_ref[g, :, 0, :]
            sg = jnp.sum(pv, axis=0, keepdims=True)
            inv = 1.0 / jnp.maximum(jnp.sqrt(sg), EPS)
            inv_ref[g, :, :] = jnp.broadcast_to(inv, (8, L))

    return pl.pallas_call(
        body,
        grid=(),
        in_specs=[in_spec],
        out_specs=out_spec,
        out_shape=out_sds,
        interpret=interpret,
    )(p)


def _final_sc(yf):
    mesh = plsc.VectorSubcoreMesh(core_axis_name="c", subcore_axis_name="s")
    out_sds = jax.ShapeDtypeStruct((G, N2, L, L), jnp.float32)
    nblocks = G * N2

    @functools.partial(
        pl.kernel,
        out_type=out_sds,
        mesh=mesh,
        scratch_types=[
            pltpu.VMEM((L, L), jnp.float32),
            pltpu.VMEM((L,), jnp.int32),
        ],
        compiler_params=_sc_params(),
    )
    def k(y_hbm, o_hbm, rowsbuf, idxv):
        c = lax.axis_index("c")
        t = lax.axis_index("s")
        wid = c * 16 + t
        iota = lax.iota(jnp.int32, 16)

        @pl.loop(wid, nblocks, step=32)
        def _(bb):
            g = bb // N2
            kb = bb - g * N2
            for jg in range(L // 16):
                jj = jg * 16 + iota
                idxv[pl.ds(jg * 16, 16)] = (
                    g * OUTP + N2P * jj + (3 * kb + jj) % N2)
            pltpu.sync_copy(y_hbm.at[idxv], rowsbuf)
            pltpu.sync_copy(rowsbuf, o_hbm.at[g, kb])

    return k(yf)


def _tc5(p2, inv, interpret=False):
    in_spec = pl.BlockSpec((1, 1, L, L), lambda g, kb: (g, kb, 0, 0))
    inv_spec = pl.BlockSpec((1, 8, L), lambda g, kb: (g, 0, 0))
    out_spec = pl.BlockSpec((L, L), lambda g, kb: (g, kb))
    out_sds = jax.ShapeDtypeStruct((B, OUT), jnp.float32)

    def body(p_ref, inv_ref, o_ref):
        tile = p_ref[0, 0, :, :]
        invrow = inv_ref[0, 0:1, :]
        o_ref[...] = jnp.transpose(tile * invrow, (1, 0))

    return pl.pallas_call(
        body,
        grid=(G, N2),
        in_specs=[in_spec, inv_spec],
        out_specs=out_spec,
        out_shape=out_sds,
        interpret=interpret,
    )(p2, inv)


def kernel(x1, x2, s1, s2, h1, h2):
    x1t = jnp.transpose(x1)
    x2t = jnp.transpose(x2)
    h1 = h1.astype(jnp.int32)
    h2 = h2.astype(jnp.int32)

    S1, S2 = _sketch_sc(x1t, s1, h1, x2t, s2, h2)
    S1v = S1.reshape(G, N1, N2P, L)
    S2v = S2.reshape(G, N1, N2P, L)
    f1r, f1i, f2r, f2i = _tc1(S1v, S2v)
    rr, ri = _tc2(f1r, f1i, f2r, f2i)
    y, p = _tc3(rr, ri)
    inv = _tc4(p)
    p2 = _final_sc(y.reshape(G * OUTP, L))
    return _tc5(p2, inv)
